# Initial kernel scaffold; baseline (speedup 1.0000x reference)
import jax
import jax.numpy as jnp
from jax.experimental import pallas as pl


def kernel(click_article_id, hist_article_ids, user_env, table_count, table_click, table_env):
    raise NotImplementedError("write your pallas kernel here")


# speedup vs baseline: 5.9813x; 5.9813x over previous
import functools

import jax
import jax.numpy as jnp
from jax import lax
from jax.experimental import pallas as pl
from jax.experimental.pallas import tpu as pltpu
from jax.experimental.pallas import tpu_sc as plsc

B = 4096
H = 50
D = 64
NC = 2
NS = 16
NW = NC * NS
BPW = B // NW
NBUF = 5
L = 16
DC = D // L

_MAGIC = 0x5F3759DF


def _inv_norm(s):
    sv = jnp.full((L,), s, jnp.float32)
    i = plsc.bitcast(sv, jnp.int32)
    y = plsc.bitcast(jnp.int32(_MAGIC) - (i >> 1), jnp.float32)
    for _ in range(3):
        y = y * (1.5 - 0.5 * sv * y * y)
    return jnp.minimum(y, 1e12)


def _make_sc_kernel():
    mesh = plsc.VectorSubcoreMesh(core_axis_name="c", subcore_axis_name="s")

    @functools.partial(
        pl.kernel,
        out_type=jax.ShapeDtypeStruct((B, 3 * D), jnp.float32),
        mesh=mesh,
        compiler_params=pltpu.CompilerParams(needs_layout_passes=False,
                                             use_tc_tiling_on_sc=False),
        scratch_types=[
            pltpu.VMEM((BPW,), jnp.int32),
            pltpu.VMEM((BPW,), jnp.int32),
            pltpu.VMEM((H, BPW), jnp.int32),
            pltpu.VMEM((BPW, D), jnp.float32),
            pltpu.VMEM((BPW, D), jnp.float32),
            pltpu.VMEM((BPW, D), jnp.float32),
            pltpu.VMEM((BPW, 3 * D), jnp.float32),
        ]
        + [pltpu.VMEM((BPW, D), jnp.float32) for _ in range(NBUF)]
        + [pltpu.SemaphoreType.DMA, pltpu.SemaphoreType.DMA]
        + [pltpu.SemaphoreType.DMA for _ in range(NBUF)],
    )
    def k(click_hbm, hist_hbm, env_hbm, tcount_hbm, tclick_hbm, tenv_hbm,
          out_hbm, idx_click, idx_env, idx_hist, click_rows, env_rows,
          acc, stage, b0, b1, b2, b3, b4, sem_c, sem_e, s0, s1, s2, s3, s4):
        bufs = (b0, b1, b2, b3, b4)
        sems = (s0, s1, s2, s3, s4)
        wid = lax.axis_index("c") * NS + lax.axis_index("s")
        base = wid * BPW

        pltpu.sync_copy(click_hbm.at[pl.ds(base, BPW)], idx_click)
        pltpu.sync_copy(env_hbm.at[pl.ds(base, BPW)], idx_env)
        pltpu.sync_copy(hist_hbm.at[:, pl.ds(base, BPW)], idx_hist)

        click_dma = pltpu.async_copy(tclick_hbm.at[idx_click], click_rows,
                                     sem_c)
        env_dma = pltpu.async_copy(tenv_hbm.at[idx_env], env_rows, sem_e)
        for b in range(NBUF):
            pltpu.async_copy(tcount_hbm.at[idx_hist.at[b]], bufs[b], sems[b])

        zero = jnp.zeros((L,), jnp.float32)

        def zrow(r, c):
            for j in range(DC):
                acc[r, pl.ds(j * L, L)] = zero
            return c

        lax.fori_loop(0, BPW, zrow, 0)

        def addrow(buf):
            def body(r, c):
                for j in range(DC):
                    plsc.addupdate(acc.at[r, pl.ds(j * L, L)],
                                   buf[r, pl.ds(j * L, L)])
                return c
            return body

        def outer(g, c):
            for b in range(NBUF):
                h = g * NBUF + b
                pltpu.make_async_copy(tcount_hbm.at[idx_hist.at[0]], bufs[b],
                                      sems[b]).wait()
                lax.fori_loop(0, BPW, addrow(bufs[b]), 0)

                @pl.when(h + NBUF < H)
                def _():
                    pltpu.async_copy(tcount_hbm.at[idx_hist.at[h + NBUF]],
                                     bufs[b], sems[b])
            return c

        lax.fori_loop(0, H // NBUF, outer, 0)

        def norm_rows(src, off):
            def body(r, c):
                vs = [src[r, pl.ds(j * L, L)] for j in range(DC)]
                s16 = vs[0] * vs[0]
                for j in range(1, DC):
                    s16 = s16 + vs[j] * vs[j]
                y = _inv_norm(jnp.sum(s16))
                for j in range(DC):
                    stage[r, pl.ds(off + j * L, L)] = vs[j] * y
                return c
            return body

        click_dma.wait()
        lax.fori_loop(0, BPW, norm_rows(click_rows, 0), 0)
        lax.fori_loop(0, BPW, norm_rows(acc, D), 0)

        env_dma.wait()

        def envrow(r, c):
            for j in range(DC):
                stage[r, pl.ds(2 * D + j * L, L)] = env_rows[r,
                                                             pl.ds(j * L, L)]
            return c

        lax.fori_loop(0, BPW, envrow, 0)

        pltpu.sync_copy(stage, out_hbm.at[pl.ds(base, BPW)])

    return k


_sc_kernel = _make_sc_kernel()


def kernel(click_article_id, hist_article_ids, user_env,
           table_count, table_click, table_env):
    ci = click_article_id.astype(jnp.int32)
    ui = user_env.astype(jnp.int32)
    hi = hist_article_ids.astype(jnp.int32).T
    return _sc_kernel(ci, hi, ui, table_count, table_click, table_env)
